# baseline (speedup 1.0000x reference)
import functools

import jax
import jax.numpy as jnp
from jax.experimental import pallas as pl
from jax.experimental.pallas import tpu as pltpu

_PATCH = 5
_D_PAD = 128


def _unfold_patches(x, k):
    C, H, W = x.shape[1], x.shape[2], x.shape[3]
    oh, ow = H - k + 1, W - k + 1
    pats = jnp.stack(
        [x[0, :, i:i + oh, j:j + ow] for i in range(k) for j in range(k)],
        axis=1,
    )
    return pats.reshape(C * k * k, oh * ow).T


def _nn_body(m_valid, n_total, x_ref, y_ref, loss_ref, idx_ref,
             best_d, best_i, loss_acc):
    i = pl.program_id(0)
    j = pl.program_id(1)
    ni = pl.num_programs(0)
    nj = pl.num_programs(1)

    x = x_ref[...]
    y = y_ref[...]
    mb = y.shape[0]

    xy = jax.lax.dot_general(
        x, y, (((1,), (1,)), ((), ())), preferred_element_type=jnp.float32)
    x2 = jnp.sum(x * x, axis=1, keepdims=True)
    y2 = jnp.sum(y * y, axis=1).reshape(1, mb)
    d = x2 - 2.0 * xy + y2

    col = j * mb + jax.lax.broadcasted_iota(jnp.int32, (1, mb), 1)
    d = jnp.where(col < m_valid, d, jnp.inf)

    bm = jnp.min(d, axis=1, keepdims=True)
    cand = jnp.min(
        jnp.where(d == bm, col, jnp.int32(2**31 - 1)),
        axis=1, keepdims=True)

    @pl.when(j == 0)
    def _init():
        best_d[...] = bm
        best_i[...] = cand

    @pl.when(j > 0)
    def _merge():
        upd = bm < best_d[...]
        best_d[...] = jnp.where(upd, bm, best_d[...])
        best_i[...] = jnp.where(upd, cand, best_i[...])

    @pl.when(jnp.logical_and(i == 0, j == 0))
    def _zero_loss():
        loss_acc[...] = jnp.zeros_like(loss_acc)

    @pl.when(j == nj - 1)
    def _flush():
        idx_ref[...] = best_i[...]
        loss_acc[...] += jnp.sum(best_d[...], axis=(0, 1), keepdims=True)

        @pl.when(i == ni - 1)
        def _loss():
            loss_ref[...] = loss_acc[...] / jnp.float32(n_total)


@jax.jit
def kernel(crop, original_image):
    X = _unfold_patches(crop, _PATCH)
    Y = _unfold_patches(original_image, _PATCH)
    n, dfeat = X.shape
    m = Y.shape[0]

    nb, mb = 720, 512
    m_pad = pl.cdiv(m, mb) * mb
    Xp = jnp.pad(X, ((0, 0), (0, _D_PAD - dfeat)))
    Yp = jnp.pad(Y, ((0, m_pad - m), (0, _D_PAD - dfeat)))

    grid = (n // nb, m_pad // mb)
    loss2d, idx2d = pl.pallas_call(
        functools.partial(_nn_body, m, n),
        grid=grid,
        in_specs=[
            pl.BlockSpec((nb, _D_PAD), lambda i, j: (i, 0)),
            pl.BlockSpec((mb, _D_PAD), lambda i, j: (j, 0)),
        ],
        out_specs=[
            pl.BlockSpec((1, 1), lambda i, j: (0, 0)),
            pl.BlockSpec((nb, 1), lambda i, j: (i, 0)),
        ],
        out_shape=[
            jax.ShapeDtypeStruct((1, 1), jnp.float32),
            jax.ShapeDtypeStruct((n, 1), jnp.int32),
        ],
        scratch_shapes=[
            pltpu.VMEM((nb, 1), jnp.float32),
            pltpu.VMEM((nb, 1), jnp.int32),
            pltpu.VMEM((1, 1), jnp.float32),
        ],
        compiler_params=pltpu.CompilerParams(
            dimension_semantics=("arbitrary", "arbitrary")),
    )(Xp, Yp)
    return loss2d[0, 0], idx2d[:, 0]


# speedup vs baseline: 1.2387x; 1.2387x over previous
import functools

import jax
import jax.numpy as jnp
from jax.experimental import pallas as pl
from jax.experimental.pallas import tpu as pltpu

_PATCH = 5
_D_PAD = 128
_LANES = 128


def _unfold_patches(x, k):
    C, H, W = x.shape[1], x.shape[2], x.shape[3]
    oh, ow = H - k + 1, W - k + 1
    pats = jnp.stack(
        [x[0, :, i:i + oh, j:j + ow] for i in range(k) for j in range(k)],
        axis=1,
    )
    return pats.reshape(C * k * k, oh * ow).T


def _nn_body(n_total, xn_ref, y_ref, x2_ref, y2_ref, loss_ref, idx_ref,
             acc_d, acc_i, loss_acc):
    i = pl.program_id(0)
    j = pl.program_id(1)
    ni = pl.num_programs(0)
    nj = pl.num_programs(1)

    xn = xn_ref[...]
    y = y_ref[...]
    x2 = x2_ref[...]
    y2 = y2_ref[...]
    mb = y.shape[0]

    @pl.when(j == 0)
    def _init():
        acc_d[...] = jnp.full(acc_d.shape, jnp.inf, jnp.float32)
        acc_i[...] = jnp.zeros(acc_i.shape, jnp.float32)

    xy2 = jax.lax.dot_general(
        xn, y, (((1,), (1,)), ((), ())), preferred_element_type=jnp.float32)

    acc = acc_d[...]
    idx = acc_i[...]
    lane = jax.lax.broadcasted_iota(
        jnp.int32, (1, _LANES), 1).astype(jnp.float32)
    for k in range(mb // _LANES):
        dk = (x2 + xy2[:, k * _LANES:(k + 1) * _LANES]) \
            + y2[:, k * _LANES:(k + 1) * _LANES]
        col = (j * mb + k * _LANES).astype(jnp.float32) + lane
        upd = dk < acc
        acc = jnp.where(upd, dk, acc)
        idx = jnp.where(upd, col, idx)
    acc_d[...] = acc
    acc_i[...] = idx

    @pl.when(jnp.logical_and(i == 0, j == 0))
    def _zero_loss():
        loss_acc[...] = jnp.zeros_like(loss_acc)

    @pl.when(j == nj - 1)
    def _flush():
        bm = jnp.min(acc, axis=1, keepdims=True)
        cand = jnp.min(
            jnp.where(acc == bm, idx, jnp.float32(3.0e8)),
            axis=1, keepdims=True)
        idx_ref[...] = cand.astype(jnp.int32)
        loss_acc[...] += jnp.sum(bm, axis=(0, 1), keepdims=True)

        @pl.when(i == ni - 1)
        def _loss():
            loss_ref[...] = loss_acc[...] / jnp.float32(n_total)


@jax.jit
def kernel(crop, original_image):
    X = _unfold_patches(crop, _PATCH)
    Y = _unfold_patches(original_image, _PATCH)
    n, dfeat = X.shape
    m = Y.shape[0]

    nb, mb = 720, 512
    m_pad = pl.cdiv(m, mb) * mb
    x2 = jnp.sum(X ** 2, axis=1, keepdims=True)
    y2 = jnp.sum(Y.T ** 2, axis=0, keepdims=True)
    y2p = jnp.pad(y2, ((0, 0), (0, m_pad - m)), constant_values=jnp.inf)
    Xn = jnp.pad(X * jnp.float32(-2.0), ((0, 0), (0, _D_PAD - dfeat)))
    Yp = jnp.pad(Y, ((0, m_pad - m), (0, _D_PAD - dfeat)))

    grid = (n // nb, m_pad // mb)
    loss2d, idx2d = pl.pallas_call(
        functools.partial(_nn_body, n),
        grid=grid,
        in_specs=[
            pl.BlockSpec((nb, _D_PAD), lambda i, j: (i, 0)),
            pl.BlockSpec((mb, _D_PAD), lambda i, j: (j, 0)),
            pl.BlockSpec((nb, 1), lambda i, j: (i, 0)),
            pl.BlockSpec((1, mb), lambda i, j: (0, j)),
        ],
        out_specs=[
            pl.BlockSpec((1, 1), lambda i, j: (0, 0)),
            pl.BlockSpec((nb, 1), lambda i, j: (i, 0)),
        ],
        out_shape=[
            jax.ShapeDtypeStruct((1, 1), jnp.float32),
            jax.ShapeDtypeStruct((n, 1), jnp.int32),
        ],
        scratch_shapes=[
            pltpu.VMEM((nb, _LANES), jnp.float32),
            pltpu.VMEM((nb, _LANES), jnp.float32),
            pltpu.VMEM((1, 1), jnp.float32),
        ],
        compiler_params=pltpu.CompilerParams(
            dimension_semantics=("arbitrary", "arbitrary")),
    )(Xn, Yp, x2, y2p)
    return loss2d[0, 0], idx2d[:, 0]


# speedup vs baseline: 1.8582x; 1.5001x over previous
import functools

import jax
import jax.numpy as jnp
from jax.experimental import pallas as pl
from jax.experimental.pallas import tpu as pltpu

_PATCH = 5
_D_PAD = 75
_LANES = 128


def _unfold_patches(x, k):
    C, H, W = x.shape[1], x.shape[2], x.shape[3]
    oh, ow = H - k + 1, W - k + 1
    pats = jnp.stack(
        [x[0, :, i:i + oh, j:j + ow] for i in range(k) for j in range(k)],
        axis=1,
    )
    return pats.reshape(C * k * k, oh * ow).T


def _nn_body(n_total, xn_ref, y_ref, x2_ref, y2_ref, loss_ref, idx_ref,
             loss_acc):
    i = pl.program_id(0)
    ni = pl.num_programs(0)

    xn = xn_ref[...]
    x2 = x2_ref[...]
    nb = xn.shape[0]
    m_pad = y_ref.shape[0]
    n_chunks = m_pad // _LANES

    lane = jax.lax.broadcasted_iota(
        jnp.int32, (1, _LANES), 1).astype(jnp.float32)

    acc = None
    idx = None
    for k in range(n_chunks):
        yk = y_ref[k * _LANES:(k + 1) * _LANES, :]
        y2k = y2_ref[:, k * _LANES:(k + 1) * _LANES]
        xyk = jax.lax.dot_general(
            xn, yk, (((1,), (1,)), ((), ())),
            preferred_element_type=jnp.float32)
        dk = (x2 + xyk) + y2k
        col = jnp.float32(k * _LANES) + lane
        if acc is None:
            acc = dk
            idx = col + jnp.zeros((nb, _LANES), jnp.float32)
        else:
            upd = dk < acc
            acc = jnp.where(upd, dk, acc)
            idx = jnp.where(upd, col, idx)

    bm = jnp.min(acc, axis=1, keepdims=True)
    cand = jnp.min(
        jnp.where(acc == bm, idx, jnp.float32(3.0e8)),
        axis=1, keepdims=True)
    idx_ref[...] = cand.astype(jnp.int32)

    @pl.when(i == 0)
    def _zero_loss():
        loss_acc[...] = jnp.zeros_like(loss_acc)

    loss_acc[...] += jnp.sum(bm, axis=(0, 1), keepdims=True)

    @pl.when(i == ni - 1)
    def _loss():
        loss_ref[...] = loss_acc[...] / jnp.float32(n_total)


@jax.jit
def kernel(crop, original_image):
    X = _unfold_patches(crop, _PATCH)
    Y = _unfold_patches(original_image, _PATCH)
    n, dfeat = X.shape
    m = Y.shape[0]

    nb = 360
    m_pad = pl.cdiv(m, _LANES) * _LANES
    x2 = jnp.sum(X ** 2, axis=1, keepdims=True)
    y2 = jnp.sum(Y.T ** 2, axis=0, keepdims=True)
    y2p = jnp.pad(y2, ((0, 0), (0, m_pad - m)), constant_values=jnp.inf)
    Xn = jnp.pad(X * jnp.float32(-2.0), ((0, 0), (0, _D_PAD - dfeat)))
    Yp = jnp.pad(Y, ((0, m_pad - m), (0, _D_PAD - dfeat)))

    grid = (n // nb,)
    loss2d, idx2d = pl.pallas_call(
        functools.partial(_nn_body, n),
        grid=grid,
        in_specs=[
            pl.BlockSpec((nb, _D_PAD), lambda i: (i, 0)),
            pl.BlockSpec((m_pad, _D_PAD), lambda i: (0, 0)),
            pl.BlockSpec((nb, 1), lambda i: (i, 0)),
            pl.BlockSpec((1, m_pad), lambda i: (0, 0)),
        ],
        out_specs=[
            pl.BlockSpec((1, 1), lambda i: (0, 0)),
            pl.BlockSpec((nb, 1), lambda i: (i, 0)),
        ],
        out_shape=[
            jax.ShapeDtypeStruct((1, 1), jnp.float32),
            jax.ShapeDtypeStruct((n, 1), jnp.int32),
        ],
        scratch_shapes=[
            pltpu.VMEM((1, 1), jnp.float32),
        ],
        compiler_params=pltpu.CompilerParams(
            dimension_semantics=("arbitrary",)),
    )(Xn, Yp, x2, y2p)
    return loss2d[0, 0], idx2d[:, 0]


# speedup vs baseline: 2.0378x; 1.0967x over previous
import functools

import jax
import jax.numpy as jnp
from jax.experimental import pallas as pl
from jax.experimental.pallas import tpu as pltpu

_PATCH = 5
_FEAT = 75
_LANES = 128


def _unfold_t(x, k):
    C, H, W = x.shape[1], x.shape[2], x.shape[3]
    oh, ow = H - k + 1, W - k + 1
    pats = jnp.stack(
        [x[0, :, i:i + oh, j:j + ow] for i in range(k) for j in range(k)],
        axis=1,
    )
    return pats.reshape(C * k * k, oh * ow)


def _nn_body(n_total, xn_ref, y_ref, x2_ref, y2_ref, loss_ref, idx_ref,
             loss_acc):
    i = pl.program_id(0)
    ni = pl.num_programs(0)

    nb = idx_ref.shape[0]
    xn = xn_ref[0]
    x2 = x2_ref[pl.ds(i * nb, nb), :]
    m_pad = y_ref.shape[1]
    n_chunks = m_pad // _LANES

    lane = jax.lax.broadcasted_iota(
        jnp.int32, (1, _LANES), 1).astype(jnp.float32)

    acc = None
    idx = None
    for k in range(n_chunks):
        yk = y_ref[:, k * _LANES:(k + 1) * _LANES]
        y2k = y2_ref[:, k * _LANES:(k + 1) * _LANES]
        xyk = jax.lax.dot_general(
            xn, yk, (((0,), (0,)), ((), ())),
            preferred_element_type=jnp.float32)
        dk = (x2 + xyk) + y2k
        col = jnp.float32(k * _LANES) + lane
        if acc is None:
            acc = dk
            idx = col + jnp.zeros((nb, _LANES), jnp.float32)
        else:
            upd = dk < acc
            acc = jnp.where(upd, dk, acc)
            idx = jnp.where(upd, col, idx)

    bm = jnp.min(acc, axis=1, keepdims=True)
    cand = jnp.min(
        jnp.where(acc == bm, idx, jnp.float32(3.0e8)),
        axis=1, keepdims=True)
    idx_ref[...] = cand.astype(jnp.int32)

    @pl.when(i == 0)
    def _zero_loss():
        loss_acc[...] = jnp.zeros_like(loss_acc)

    loss_acc[...] += jnp.sum(bm, axis=(0, 1), keepdims=True)

    @pl.when(i == ni - 1)
    def _loss():
        loss_ref[...] = loss_acc[...] / jnp.float32(n_total)


@jax.jit
def kernel(crop, original_image):
    Xt = _unfold_t(crop, _PATCH)
    Yt = _unfold_t(original_image, _PATCH)
    n = Xt.shape[1]
    m = Yt.shape[1]

    nb = 360
    m_pad = pl.cdiv(m, _LANES) * _LANES
    x2 = jnp.sum(Xt ** 2, axis=0)[:, None]
    y2 = jnp.sum(Yt ** 2, axis=0, keepdims=True)
    y2p = jnp.pad(y2, ((0, 0), (0, m_pad - m)), constant_values=jnp.inf)
    Xn = (Xt * jnp.float32(-2.0)).reshape(_FEAT, n // nb, nb).swapaxes(0, 1)
    Yp = jnp.pad(Yt, ((0, 0), (0, m_pad - m)))

    grid = (n // nb,)
    loss2d, idx2d = pl.pallas_call(
        functools.partial(_nn_body, n),
        grid=grid,
        in_specs=[
            pl.BlockSpec((1, _FEAT, nb), lambda i: (i, 0, 0)),
            pl.BlockSpec((_FEAT, m_pad), lambda i: (0, 0)),
            pl.BlockSpec((n, 1), lambda i: (0, 0)),
            pl.BlockSpec((1, m_pad), lambda i: (0, 0)),
        ],
        out_specs=[
            pl.BlockSpec((1, 1), lambda i: (0, 0)),
            pl.BlockSpec((nb, 1), lambda i: (i, 0)),
        ],
        out_shape=[
            jax.ShapeDtypeStruct((1, 1), jnp.float32),
            jax.ShapeDtypeStruct((n, 1), jnp.int32),
        ],
        scratch_shapes=[
            pltpu.VMEM((1, 1), jnp.float32),
        ],
        compiler_params=pltpu.CompilerParams(
            dimension_semantics=("arbitrary",)),
    )(Xn, Yp, x2, y2p)
    return loss2d[0, 0], idx2d[:, 0]
